# Initial kernel scaffold; baseline (speedup 1.0000x reference)
import functools

import jax
import jax.numpy as jnp
from jax import lax
from jax.experimental import pallas as pl
from jax.experimental.pallas import tpu as pltpu
from jax.experimental.pallas import tpu_sc as plsc

N = 10000
D = 128
E = 320000
TELEPORT = 0.1

NC, NS, L = 2, 16, 16
NW = NC * NS
EPW = E // NW
CHUNK = 80
NCHUNK = EPW // CHUNK
ROWS_PT = N // NS
ZROWS = 125


def _sc_body(p_hbm, src_hbm, dst_hbm, w_hbm, out_hbm,
             acc, idx_s, idx_d, wv, rows, sem):
    c = lax.axis_index("c")
    s = lax.axis_index("s")
    wid = c * NS + s

    zero = jnp.zeros((L,), jnp.float32)

    def zrow(r, carry):
        for k in range(D // L):
            rows[r, pl.ds(k * L, L)] = zero
        return carry

    lax.fori_loop(0, CHUNK, zrow, 0)
    zslice = rows.at[pl.ds(0, CHUNK)]
    for i in range(ROWS_PT // CHUNK):
        pltpu.sync_copy(zslice, acc.at[pl.ds(s * ROWS_PT + i * CHUNK, CHUNK)])
    rem = ROWS_PT - (ROWS_PT // CHUNK) * CHUNK
    if rem:
        pltpu.sync_copy(rows.at[pl.ds(0, rem)],
                        acc.at[pl.ds(s * ROWS_PT + (ROWS_PT // CHUNK) * CHUNK,
                                     rem)])
    plsc.subcore_barrier()

    ebase = wid * EPW

    def chunk_body(t, carry):
        off = ebase + t * CHUNK
        pltpu.sync_copy(src_hbm.at[pl.ds(off, CHUNK)], idx_s)
        pltpu.sync_copy(dst_hbm.at[pl.ds(off, CHUNK)], idx_d)
        pltpu.sync_copy(w_hbm.at[pl.ds(off, CHUNK)], wv)
        pltpu.async_copy(p_hbm.at[idx_s], rows, sem).wait()
        for e in range(CHUNK):
            we = jnp.full((L,), wv[e], jnp.float32)
            for k in range(D // L):
                sl = pl.ds(k * L, L)
                rows[e, sl] = rows[e, sl] * we
        pltpu.sync_copy(rows, acc.at[idx_d], add=True)
        return carry

    lax.fori_loop(0, NCHUNK, chunk_body, 0)
    plsc.subcore_barrier()

    nfull = ROWS_PT // CHUNK
    for i in range(nfull):
        sl = pl.ds(s * ROWS_PT + i * CHUNK, CHUNK)
        pltpu.sync_copy(acc.at[sl], out_hbm.at[c].at[sl])
    rem = ROWS_PT - nfull * CHUNK
    if rem:
        sl = pl.ds(s * ROWS_PT + nfull * CHUNK, rem)
        pltpu.sync_copy(acc.at[sl], out_hbm.at[c].at[sl])


_sc_spmm = functools.partial(
    pl.kernel,
    out_type=jax.ShapeDtypeStruct((NC, N, D), jnp.float32),
    mesh=plsc.VectorSubcoreMesh(core_axis_name="c", subcore_axis_name="s"),
    scratch_types=[
        pltpu.VMEM_SHARED((N, D), jnp.float32),
        pltpu.VMEM((CHUNK,), jnp.int32),
        pltpu.VMEM((CHUNK,), jnp.int32),
        pltpu.VMEM((CHUNK,), jnp.float32),
        pltpu.VMEM((CHUNK, D), jnp.float32),
        pltpu.SemaphoreType.DMA,
    ],
)(_sc_body)


def _combine_body(p0_ref, p1_ref, f_ref, o_ref):
    o_ref[...] = ((1.0 - TELEPORT) * (p0_ref[...] + p1_ref[...])
                  + TELEPORT * f_ref[...])


_combine = pl.pallas_call(
    _combine_body,
    out_shape=jax.ShapeDtypeStruct((N, D), jnp.float32),
    grid=(10,),
    in_specs=[pl.BlockSpec((N // 10, D), lambda i: (i, 0))] * 3,
    out_specs=pl.BlockSpec((N // 10, D), lambda i: (i, 0)),
)


@jax.jit
def kernel(propagated_features, features, edge_index, edge_weight):
    p = jnp.squeeze(propagated_features, axis=0)
    f = jnp.squeeze(features, axis=0)
    src = edge_index[0].astype(jnp.int32)
    dst = edge_index[1].astype(jnp.int32)
    w = edge_weight.astype(jnp.float32)
    partial = _sc_spmm(p, src, dst, w)
    out = _combine(partial[0], partial[1], f)
    return out[None]


# speedup vs baseline: 4.1326x; 4.1326x over previous
import functools

import jax
import jax.numpy as jnp
from jax import lax
from jax.experimental import pallas as pl
from jax.experimental.pallas import tpu as pltpu
from jax.experimental.pallas import tpu_sc as plsc

N = 10000
D = 128
E = 320000
TELEPORT = 0.1

NC, NS, L = 2, 16, 16
NW = NC * NS
EPW = E // NW
CHUNK = 80
NCHUNK = EPW // CHUNK
N_PAD = 10240
ROWS_PT = N_PAD // NS


def _sc_body(p_hbm, src_hbm, dst_hbm, w_hbm, out_hbm,
             acc, idx_s, idx_d, wv, rows, sem):
    c = lax.axis_index("c")
    s = lax.axis_index("s")
    wid = c * NS + s

    zero = jnp.zeros((L,), jnp.float32)

    def zrow(r, carry):
        for k in range(D // L):
            rows[r, pl.ds(k * L, L)] = zero
        return carry

    lax.fori_loop(0, CHUNK, zrow, 0)
    for i in range(ROWS_PT // CHUNK):
        pltpu.sync_copy(rows, acc.at[pl.ds(s * ROWS_PT + i * CHUNK, CHUNK)])
    plsc.subcore_barrier()

    ebase = wid * EPW

    def chunk_body(t, carry):
        off = ebase + t * CHUNK
        pltpu.sync_copy(src_hbm.at[pl.ds(off, CHUNK)], idx_s)
        pltpu.sync_copy(dst_hbm.at[pl.ds(off, CHUNK)], idx_d)
        pltpu.sync_copy(w_hbm.at[pl.ds(off, CHUNK)], wv)
        pltpu.async_copy(p_hbm.at[idx_s], rows, sem).wait()
        for g in range(CHUNK // L):
            wreg = wv[pl.ds(g * L, L)]
            for j in range(L):
                e = g * L + j
                we = jnp.full((L,), wreg[j], jnp.float32)
                for k in range(D // L):
                    sl = pl.ds(k * L, L)
                    rows[e, sl] = rows[e, sl] * we
        pltpu.sync_copy(rows, acc.at[idx_d], add=True)
        return carry

    lax.fori_loop(0, NCHUNK, chunk_body, 0)
    plsc.subcore_barrier()

    for i in range(ROWS_PT // CHUNK):
        sl = pl.ds(s * ROWS_PT + i * CHUNK, CHUNK)
        pltpu.sync_copy(acc.at[sl], out_hbm.at[c].at[sl])


_sc_spmm = functools.partial(
    pl.kernel,
    out_type=jax.ShapeDtypeStruct((NC, N_PAD, D), jnp.float32),
    mesh=plsc.VectorSubcoreMesh(core_axis_name="c", subcore_axis_name="s"),
    scratch_types=[
        pltpu.VMEM_SHARED((N_PAD, D), jnp.float32),
        pltpu.VMEM((CHUNK,), jnp.int32),
        pltpu.VMEM((CHUNK,), jnp.int32),
        pltpu.VMEM((CHUNK,), jnp.float32),
        pltpu.VMEM((CHUNK, D), jnp.float32),
        pltpu.SemaphoreType.DMA,
    ],
)(_sc_body)


def _combine_body(p0_ref, p1_ref, f_ref, o_ref):
    o_ref[...] = ((1.0 - TELEPORT) * (p0_ref[...] + p1_ref[...])
                  + TELEPORT * f_ref[...])


_combine = pl.pallas_call(
    _combine_body,
    out_shape=jax.ShapeDtypeStruct((N, D), jnp.float32),
    grid=(10,),
    in_specs=[pl.BlockSpec((N // 10, D), lambda i: (i, 0))] * 3,
    out_specs=pl.BlockSpec((N // 10, D), lambda i: (i, 0)),
)


@jax.jit
def kernel(propagated_features, features, edge_index, edge_weight):
    p = jnp.squeeze(propagated_features, axis=0)
    f = jnp.squeeze(features, axis=0)
    src = edge_index[0].astype(jnp.int32)
    dst = edge_index[1].astype(jnp.int32)
    w = edge_weight.astype(jnp.float32)
    partial = _sc_spmm(p, src, dst, w)
    out = _combine(partial[0, :N], partial[1, :N], f)
    return out[None]
